# Initial kernel scaffold; baseline (speedup 1.0000x reference)
import jax
import jax.numpy as jnp
from jax.experimental import pallas as pl


def kernel(seq_len, table):
    raise NotImplementedError("write your pallas kernel here")


# speedup vs baseline: 42.5210x; 42.5210x over previous
import math

import jax
import jax.numpy as jnp
import numpy as np
from jax import lax
from jax.experimental import pallas as pl
from jax.experimental.pallas import tpu as pltpu
from jax.experimental.pallas import tpu_sc as plsc

NUM_HEADS = 16
NUM_BUCKETS = 32
MAX_DISTANCE = 256
S = 2048
L = 16
LINE_PAD = 4112
RSTRIDE = 4096
ROWS_PER_W = S // 2
GROUP = 8


def _bucket_thresholds():
    a = np.arange(S + 64, dtype=np.int64)
    x = a.astype(np.float32) / np.float32(8.0) + np.float32(1e-6)
    lp = np.log(x) / np.float32(math.log(MAX_DISTANCE / 8.0)) * np.float32(8.0)
    lpi = np.minimum((np.float32(8.0) + lp).astype(np.int32), 15)
    bab = np.where(a < 8, a, lpi).astype(np.int32)
    return [int(np.argmax(bab >= b)) for b in range(1, 16)]


_THRESHOLDS = _bucket_thresholds()


def _sc_body(table_hbm, out_hbm, table_v, line_v, shift_v, sem):
    h = lax.axis_index("s")
    half = lax.axis_index("c")
    row0 = half * ROWS_PER_W

    pltpu.sync_copy(table_hbm, table_v)

    lane = lax.iota(jnp.int32, 16)

    def line_body(c, carry):
        k = c * L + lane
        rel = k - (S - 1)
        a = jnp.abs(rel)
        bucket = jnp.where(rel > 0, NUM_BUCKETS // 2, 0)
        for t in _THRESHOLDS:
            bucket = bucket + jnp.where(a >= t, 1, 0)
        idx = bucket * NUM_HEADS + h
        line_v[pl.ds(c * L, L)] = plsc.load_gather(table_v, [idx])
        return carry

    lax.fori_loop(0, LINE_PAD // L, line_body, 0)

    def shift_body(c, carry):
        base = c * L
        kidx = base + lane
        for r in range(8):
            shift_v[pl.ds(r * RSTRIDE + base, L)] = plsc.load_gather(
                line_v, [kidx + r])
        return carry

    lax.fori_loop(0, RSTRIDE // L, shift_body, 0)

    def dma_group(g, carry):
        for b in range(GROUP):
            i = row0 + g * GROUP + b
            o = (S - 1) - i
            r = jnp.bitwise_and(o, 7)
            off = pl.multiple_of((o - r) + r * RSTRIDE, 8)
            pltpu.make_async_copy(
                shift_v.at[pl.ds(off, S)], out_hbm.at[0, h, i], sem
            ).start()

        @pl.when(g > 0)
        def _drain():
            for b in range(GROUP):
                pltpu.make_async_copy(
                    shift_v.at[pl.ds(0, S)], out_hbm.at[0, h, row0], sem
                ).wait()

        return carry

    lax.fori_loop(0, ROWS_PER_W // GROUP, dma_group, 0)

    for b in range(GROUP):
        pltpu.make_async_copy(
            shift_v.at[pl.ds(0, S)], out_hbm.at[0, h, row0], sem
        ).wait()


def kernel(seq_len, table):
    del seq_len
    table_flat = table.reshape(NUM_BUCKETS * NUM_HEADS)
    launch = pl.kernel(
        _sc_body,
        mesh=plsc.VectorSubcoreMesh(core_axis_name="c", subcore_axis_name="s"),
        compiler_params=pltpu.CompilerParams(
            needs_layout_passes=False, use_tc_tiling_on_sc=False),
        out_type=jax.ShapeDtypeStruct((1, NUM_HEADS, S, S), jnp.float32),
        scratch_types=[
            pltpu.VMEM((NUM_BUCKETS * NUM_HEADS,), jnp.float32),
            pltpu.VMEM((LINE_PAD,), jnp.float32),
            pltpu.VMEM((8 * RSTRIDE,), jnp.float32),
            pltpu.SemaphoreType.DMA,
        ],
    )
    return launch(table_flat)


# speedup vs baseline: 54.5493x; 1.2829x over previous
import functools
import math

import jax
import jax.numpy as jnp
import numpy as np
from jax import lax
from jax.experimental import pallas as pl
from jax.experimental.pallas import tpu as pltpu
from jax.experimental.pallas import tpu_sc as plsc

NUM_HEADS = 16
NUM_BUCKETS = 32
MAX_DISTANCE = 256
S = 2048
L = 16
WIDTH = 4352
HALF_W = WIDTH // 2
LINE_PAD = WIDTH + 16
ROWS_PER_TC_BLK = 64


def _bucket_thresholds():
    a = np.arange(S + 64, dtype=np.int64)
    x = a.astype(np.float32) / np.float32(8.0) + np.float32(1e-6)
    lp = np.log(x) / np.float32(math.log(MAX_DISTANCE / 8.0)) * np.float32(8.0)
    lpi = np.minimum((np.float32(8.0) + lp).astype(np.int32), 15)
    bab = np.where(a < 8, a, lpi).astype(np.int32)
    return [int(np.argmax(bab >= b)) for b in range(1, 16)]


_THRESHOLDS = _bucket_thresholds()


def _sc_lines_body(table_hbm, shift_hbm, table_v, line_v, shift_v, sem):
    h = lax.axis_index("s")
    half = lax.axis_index("c")
    k0 = half * HALF_W

    pltpu.sync_copy(table_hbm, table_v)

    lane = lax.iota(jnp.int32, 16)

    def line_body(c, carry):
        k = k0 + c * L + lane
        rel = k - (S - 1)
        a = jnp.abs(rel)
        bucket = jnp.where(rel > 0, NUM_BUCKETS // 2, 0)
        for t in _THRESHOLDS:
            bucket = bucket + jnp.where(a >= t, 1, 0)
        idx = bucket * NUM_HEADS + h
        line_v[pl.ds(k0 + c * L, L)] = plsc.load_gather(table_v, [idx])
        return carry

    lax.fori_loop(0, HALF_W // L + 1, line_body, 0)

    def shift_body(c, carry):
        base = k0 + c * L
        kidx = base + lane
        for r in range(8):
            shift_v[r, pl.ds(base - k0, L)] = plsc.load_gather(
                line_v, [kidx + 7 - r])
        return carry

    lax.fori_loop(0, HALF_W // L, shift_body, 0)

    pltpu.sync_copy(shift_v, shift_hbm.at[h, :, pl.ds(k0, HALF_W)])


def _tc_stripes_body(shift_ref, out_ref):
    blk = pl.program_id(1)
    i0 = blk * ROWS_PER_TC_BLK
    for s in range(ROWS_PER_TC_BLK // 8):
        q = (S - 8) - (i0 + s * 8)
        rem = jnp.bitwise_and(q, 127)
        qa = pl.multiple_of(q - rem, 128)
        coarse = shift_ref[0, :, pl.ds(qa, S + 128)]
        rolled = pltpu.roll(coarse, (S + 128) - rem, axis=1)
        out_ref[0, 0, pl.ds(s * 8, 8), :] = rolled[:, :S]


def kernel(seq_len, table):
    del seq_len
    table_flat = table.reshape(NUM_BUCKETS * NUM_HEADS)

    sc_launch = pl.kernel(
        _sc_lines_body,
        mesh=plsc.VectorSubcoreMesh(core_axis_name="c", subcore_axis_name="s"),
        out_type=jax.ShapeDtypeStruct((NUM_HEADS, 8, WIDTH), jnp.float32),
        compiler_params=pltpu.CompilerParams(needs_layout_passes=False),
        scratch_types=[
            pltpu.VMEM((NUM_BUCKETS * NUM_HEADS,), jnp.float32),
            pltpu.VMEM((LINE_PAD,), jnp.float32),
            pltpu.VMEM((8, HALF_W), jnp.float32),
            pltpu.SemaphoreType.DMA,
        ],
    )
    shift = sc_launch(table_flat)

    out = pl.pallas_call(
        _tc_stripes_body,
        grid=(NUM_HEADS, S // ROWS_PER_TC_BLK),
        in_specs=[
            pl.BlockSpec((1, 8, WIDTH), lambda h, b: (h, 0, 0)),
        ],
        out_specs=pl.BlockSpec(
            (1, 1, ROWS_PER_TC_BLK, S), lambda h, b: (0, h, b, 0)),
        out_shape=jax.ShapeDtypeStruct((1, NUM_HEADS, S, S), jnp.float32),
        compiler_params=pltpu.CompilerParams(
            dimension_semantics=("parallel", "arbitrary")),
    )(shift)
    return out


# speedup vs baseline: 94.9830x; 1.7412x over previous
import functools
import math

import jax
import jax.numpy as jnp
import numpy as np
from jax import lax
from jax.experimental import pallas as pl
from jax.experimental.pallas import tpu as pltpu
from jax.experimental.pallas import tpu_sc as plsc

NUM_HEADS = 16
NUM_BUCKETS = 32
MAX_DISTANCE = 256
S = 2048
L = 16
WIDTH = 4352
HALF_W = WIDTH // 2
LINE_PAD = WIDTH + 16
ROWS_PER_TC_BLK = 64


def _bucket_thresholds():
    a = np.arange(S + 64, dtype=np.int64)
    x = a.astype(np.float32) / np.float32(8.0) + np.float32(1e-6)
    lp = np.log(x) / np.float32(math.log(MAX_DISTANCE / 8.0)) * np.float32(8.0)
    lpi = np.minimum((np.float32(8.0) + lp).astype(np.int32), 15)
    bab = np.where(a < 8, a, lpi).astype(np.int32)
    return [int(np.argmax(bab >= b)) for b in range(1, 16)]


_THRESHOLDS = _bucket_thresholds()


def _sc_lines_body(table_hbm, shift_hbm, table_v, line_v, shift_v, sem):
    h = lax.axis_index("s")
    half = lax.axis_index("c")
    k0 = half * HALF_W

    pltpu.sync_copy(table_hbm, table_v)

    lane = lax.iota(jnp.int32, 16)

    def line_body(c, carry):
        k = k0 + c * L + lane
        rel = k - (S - 1)
        a = jnp.abs(rel)
        bucket = jnp.where(rel > 0, NUM_BUCKETS // 2, 0)
        for t in _THRESHOLDS:
            bucket = bucket + jnp.where(a >= t, 1, 0)
        idx = bucket * NUM_HEADS + h
        line_v[pl.ds(k0 + c * L, L)] = plsc.load_gather(table_v, [idx])
        return carry

    lax.fori_loop(0, HALF_W // L + 1, line_body, 0)

    def shift_body(c, carry):
        base = k0 + c * L
        kidx = base + lane
        for r in range(8):
            shift_v[r, pl.ds(base - k0, L)] = plsc.load_gather(
                line_v, [kidx + 7 - r])
        return carry

    lax.fori_loop(0, HALF_W // L, shift_body, 0)

    pltpu.sync_copy(shift_v, shift_hbm.at[h, :, pl.ds(k0, HALF_W)])


def _tc_stripes_body(shift_ref, out_ref, buf, sem):
    h = pl.program_id(0)
    v = pl.program_id(1)
    p = h * 16 + v
    par = jnp.bitwise_and(p, 1)
    nlast = NUM_HEADS * 16 - 1

    def _drain(par_idx, n):
        for _ in range(n):
            pltpu.make_async_copy(
                buf.at[par_idx, :, pl.ds(0, S)],
                out_ref.at[0, 0, pl.ds(0, 8), :],
                sem.at[par_idx],
            ).wait()

    @pl.when(p >= 2)
    def _():
        _drain(par, 16)

    rolled = pltpu.roll(shift_ref[0], WIDTH - 8 * v, axis=1)
    buf[par] = rolled

    for a in range(16):
        i = (S - 8) - 128 * a - 8 * v
        pltpu.make_async_copy(
            buf.at[par, :, pl.ds(128 * a, S)],
            out_ref.at[0, h, pl.ds(pl.multiple_of(i, 8), 8), :],
            sem.at[par],
        ).start()

    @pl.when(p == nlast)
    def _():
        _drain(par, 16)
        _drain(1 - par, 16)


def kernel(seq_len, table):
    del seq_len
    table_flat = table.reshape(NUM_BUCKETS * NUM_HEADS)

    sc_launch = pl.kernel(
        _sc_lines_body,
        mesh=plsc.VectorSubcoreMesh(core_axis_name="c", subcore_axis_name="s"),
        out_type=jax.ShapeDtypeStruct((NUM_HEADS, 8, WIDTH), jnp.float32),
        compiler_params=pltpu.CompilerParams(needs_layout_passes=False),
        scratch_types=[
            pltpu.VMEM((NUM_BUCKETS * NUM_HEADS,), jnp.float32),
            pltpu.VMEM((LINE_PAD,), jnp.float32),
            pltpu.VMEM((8, HALF_W), jnp.float32),
            pltpu.SemaphoreType.DMA,
        ],
    )
    shift = sc_launch(table_flat)

    out = pl.pallas_call(
        _tc_stripes_body,
        grid=(NUM_HEADS, 16),
        in_specs=[
            pl.BlockSpec((1, 8, WIDTH), lambda h, v: (h, 0, 0)),
        ],
        out_specs=pl.BlockSpec(memory_space=pl.ANY),
        out_shape=jax.ShapeDtypeStruct((1, NUM_HEADS, S, S), jnp.float32),
        scratch_shapes=[
            pltpu.VMEM((2, 8, WIDTH), jnp.float32),
            pltpu.SemaphoreType.DMA((2,)),
        ],
        compiler_params=pltpu.CompilerParams(
            dimension_semantics=("arbitrary", "arbitrary")),
    )(shift)
    return out


# speedup vs baseline: 129.7429x; 1.3660x over previous
import functools
import math

import jax
import jax.numpy as jnp
import numpy as np
from jax import lax
from jax.experimental import pallas as pl
from jax.experimental.pallas import tpu as pltpu
from jax.experimental.pallas import tpu_sc as plsc

NUM_HEADS = 16
NUM_BUCKETS = 32
MAX_DISTANCE = 256
S = 2048
L = 16
WIDTH = 4352
HALF_W = WIDTH // 2
LINE_PAD = WIDTH + 16
ROWS_PER_TC_BLK = 64


def _bucket_thresholds():
    a = np.arange(S + 64, dtype=np.int64)
    x = a.astype(np.float32) / np.float32(8.0) + np.float32(1e-6)
    lp = np.log(x) / np.float32(math.log(MAX_DISTANCE / 8.0)) * np.float32(8.0)
    lpi = np.minimum((np.float32(8.0) + lp).astype(np.int32), 15)
    bab = np.where(a < 8, a, lpi).astype(np.int32)
    return [int(np.argmax(bab >= b)) for b in range(1, 16)]


_THRESHOLDS = _bucket_thresholds()


def _sc_lines_body(table_hbm, shift_hbm, table_v, line_v, shift_v, sem):
    h = lax.axis_index("s")
    half = lax.axis_index("c")
    k0 = half * HALF_W

    pltpu.sync_copy(table_hbm, table_v)

    lane = lax.iota(jnp.int32, 16)

    def line_body(c, carry):
        k = k0 + c * L + lane
        rel = k - (S - 1)
        a = jnp.abs(rel)
        bucket = jnp.where(rel > 0, NUM_BUCKETS // 2, 0)
        for t in _THRESHOLDS:
            bucket = bucket + jnp.where(a >= t, 1, 0)
        idx = bucket * NUM_HEADS + h
        line_v[pl.ds(k0 + c * L, L)] = plsc.load_gather(table_v, [idx])
        return carry

    lax.fori_loop(0, HALF_W // L + 1, line_body, 0)

    def shift_body(c, carry):
        base = k0 + c * L
        kidx = base + lane
        for r in range(8):
            shift_v[r, pl.ds(base - k0, L)] = plsc.load_gather(
                line_v, [kidx + 7 - r])
        return carry

    lax.fori_loop(0, HALF_W // L, shift_body, 0)

    pltpu.sync_copy(shift_v, shift_hbm.at[h, :, pl.ds(k0, HALF_W)])


def _tc_stripes_body(shift_ref, out_ref, buf, sem):
    h = pl.program_id(0)
    v = pl.program_id(1)
    p = h * 16 + v
    par = jnp.bitwise_and(p, 3)
    nlast = NUM_HEADS * 16 - 1

    def _drain(par_idx, n):
        for _ in range(n):
            pltpu.make_async_copy(
                buf.at[par_idx, :, pl.ds(0, S)],
                out_ref.at[0, 0, pl.ds(0, 8), :],
                sem.at[par_idx],
            ).wait()

    rolled = pltpu.roll(shift_ref[0], WIDTH - 8 * v, axis=1)

    @pl.when(p >= 4)
    def _():
        _drain(par, 16)

    buf[par] = rolled

    for a in range(16):
        i = (S - 8) - 128 * a - 8 * v
        pltpu.make_async_copy(
            buf.at[par, :, pl.ds(128 * a, S)],
            out_ref.at[0, h, pl.ds(pl.multiple_of(i, 8), 8), :],
            sem.at[par],
        ).start()

    @pl.when(p == nlast)
    def _():
        for slot in range(4):
            _drain(slot, 16)


def kernel(seq_len, table):
    del seq_len
    table_flat = table.reshape(NUM_BUCKETS * NUM_HEADS)

    sc_launch = pl.kernel(
        _sc_lines_body,
        mesh=plsc.VectorSubcoreMesh(core_axis_name="c", subcore_axis_name="s"),
        out_type=jax.ShapeDtypeStruct((NUM_HEADS, 8, WIDTH), jnp.float32),
        compiler_params=pltpu.CompilerParams(needs_layout_passes=False),
        scratch_types=[
            pltpu.VMEM((NUM_BUCKETS * NUM_HEADS,), jnp.float32),
            pltpu.VMEM((LINE_PAD,), jnp.float32),
            pltpu.VMEM((8, HALF_W), jnp.float32),
            pltpu.SemaphoreType.DMA,
        ],
    )
    shift = sc_launch(table_flat)

    out = pl.pallas_call(
        _tc_stripes_body,
        grid=(NUM_HEADS, 16),
        in_specs=[
            pl.BlockSpec((1, 8, WIDTH), lambda h, v: (h, 0, 0)),
        ],
        out_specs=pl.BlockSpec(memory_space=pl.ANY),
        out_shape=jax.ShapeDtypeStruct((1, NUM_HEADS, S, S), jnp.float32),
        scratch_shapes=[
            pltpu.VMEM((4, 8, WIDTH), jnp.float32),
            pltpu.SemaphoreType.DMA((4,)),
        ],
        compiler_params=pltpu.CompilerParams(
            dimension_semantics=("arbitrary", "arbitrary")),
    )(shift)
    return out


# speedup vs baseline: 145.5288x; 1.1217x over previous
import functools
import math

import jax
import jax.numpy as jnp
import numpy as np
from jax import lax
from jax.experimental import pallas as pl
from jax.experimental.pallas import tpu as pltpu
from jax.experimental.pallas import tpu_sc as plsc

NUM_HEADS = 16
NUM_BUCKETS = 32
MAX_DISTANCE = 256
S = 2048
L = 16
WIDTH = 4352
HALF_W = WIDTH // 2
LINE_PAD = WIDTH + 16
ROWS_PER_TC_BLK = 64


def _bucket_thresholds():
    a = np.arange(S + 64, dtype=np.int64)
    x = a.astype(np.float32) / np.float32(8.0) + np.float32(1e-6)
    lp = np.log(x) / np.float32(math.log(MAX_DISTANCE / 8.0)) * np.float32(8.0)
    lpi = np.minimum((np.float32(8.0) + lp).astype(np.int32), 15)
    bab = np.where(a < 8, a, lpi).astype(np.int32)
    return [int(np.argmax(bab >= b)) for b in range(1, 16)]


_THRESHOLDS = _bucket_thresholds()


def _sc_lines_body(table_hbm, shift_hbm, table_v, line_v, shift_v, sem):
    h = lax.axis_index("s")
    half = lax.axis_index("c")
    k0 = half * HALF_W

    pltpu.sync_copy(table_hbm, table_v)

    lane = lax.iota(jnp.int32, 16)

    def line_body(c, carry):
        k = k0 + c * L + lane
        rel = k - (S - 1)
        a = jnp.abs(rel)
        bucket = jnp.where(rel > 0, NUM_BUCKETS // 2, 0)
        for t in _THRESHOLDS:
            bucket = bucket + jnp.where(a >= t, 1, 0)
        idx = bucket * NUM_HEADS + h
        line_v[pl.ds(k0 + c * L, L)] = plsc.load_gather(table_v, [idx])
        return carry

    lax.fori_loop(0, HALF_W // L + 1, line_body, 0)

    def shift_body(c, carry):
        base = k0 + c * L
        kidx = base + lane
        for r in range(8):
            shift_v[r, pl.ds(base - k0, L)] = plsc.load_gather(
                line_v, [kidx + 7 - r])
        return carry

    lax.fori_loop(0, HALF_W // L, shift_body, 0)

    pltpu.sync_copy(shift_v, shift_hbm.at[h, :, pl.ds(k0, HALF_W)])


def _tc_stripes_body(shift_ref, out_ref, buf, sem):
    h = pl.program_id(0)
    v = pl.program_id(1)
    p = h * 16 + v
    par = jnp.bitwise_and(p, 7)
    nlast = NUM_HEADS * 16 - 1

    def _drain(par_idx):
        pltpu.make_async_copy(
            out_ref.at[0, 1, pl.ds(0, 128), :],
            out_ref.at[0, 0, pl.ds(0, 128), :],
            sem.at[par_idx],
        ).wait()

    rolled = pltpu.roll(shift_ref[0], WIDTH - 8 * v, axis=1)

    @pl.when(p >= 8)
    def _():
        _drain(par)

    buf[par] = rolled

    for a in range(16):
        i = (S - 8) - 128 * a - 8 * v
        pltpu.make_async_copy(
            buf.at[par, :, pl.ds(128 * a, S)],
            out_ref.at[0, h, pl.ds(pl.multiple_of(i, 8), 8), :],
            sem.at[par],
        ).start()

    @pl.when(p == nlast)
    def _():
        for slot in range(8):
            _drain(slot)


def kernel(seq_len, table):
    del seq_len
    table_flat = table.reshape(NUM_BUCKETS * NUM_HEADS)

    sc_launch = pl.kernel(
        _sc_lines_body,
        mesh=plsc.VectorSubcoreMesh(core_axis_name="c", subcore_axis_name="s"),
        out_type=jax.ShapeDtypeStruct((NUM_HEADS, 8, WIDTH), jnp.float32),
        compiler_params=pltpu.CompilerParams(needs_layout_passes=False),
        scratch_types=[
            pltpu.VMEM((NUM_BUCKETS * NUM_HEADS,), jnp.float32),
            pltpu.VMEM((LINE_PAD,), jnp.float32),
            pltpu.VMEM((8, HALF_W), jnp.float32),
            pltpu.SemaphoreType.DMA,
        ],
    )
    shift = sc_launch(table_flat)

    out = pl.pallas_call(
        _tc_stripes_body,
        grid=(NUM_HEADS, 16),
        in_specs=[
            pl.BlockSpec((1, 8, WIDTH), lambda h, v: (h, 0, 0)),
        ],
        out_specs=pl.BlockSpec(memory_space=pl.ANY),
        out_shape=jax.ShapeDtypeStruct((1, NUM_HEADS, S, S), jnp.float32),
        scratch_shapes=[
            pltpu.VMEM((8, 8, WIDTH), jnp.float32),
            pltpu.SemaphoreType.DMA((8,)),
        ],
        compiler_params=pltpu.CompilerParams(
            dimension_semantics=("arbitrary", "arbitrary")),
    )(shift)
    return out


# speedup vs baseline: 146.2683x; 1.0051x over previous
import functools
import math

import jax
import jax.numpy as jnp
import numpy as np
from jax import lax
from jax.experimental import pallas as pl
from jax.experimental.pallas import tpu as pltpu
from jax.experimental.pallas import tpu_sc as plsc

NUM_HEADS = 16
NUM_BUCKETS = 32
MAX_DISTANCE = 256
S = 2048
L = 16
WIDTH = 4352
HALF_W = WIDTH // 2
LINE_PAD = WIDTH + 16
ROWS_PER_TC_BLK = 64


def _bucket_thresholds():
    a = np.arange(S + 64, dtype=np.int64)
    x = a.astype(np.float32) / np.float32(8.0) + np.float32(1e-6)
    lp = np.log(x) / np.float32(math.log(MAX_DISTANCE / 8.0)) * np.float32(8.0)
    lpi = np.minimum((np.float32(8.0) + lp).astype(np.int32), 15)
    bab = np.where(a < 8, a, lpi).astype(np.int32)
    return [int(np.argmax(bab >= b)) for b in range(1, 16)]


_THRESHOLDS = _bucket_thresholds()


def _sc_lines_body(table_hbm, shift_hbm, table_v, line_v, shift_v, sem):
    h = lax.axis_index("s")
    half = lax.axis_index("c")
    k0 = half * HALF_W

    pltpu.sync_copy(table_hbm, table_v)

    lane = lax.iota(jnp.int32, 16)

    def line_body(c, carry):
        k = k0 + c * L + lane
        rel = k - (S - 1)
        a = jnp.abs(rel)
        bucket = jnp.where(rel > 0, NUM_BUCKETS // 2, 0) + jnp.minimum(a, 7)
        for t in _THRESHOLDS[7:]:
            bucket = bucket + jnp.where(a >= t, 1, 0)
        idx = bucket * NUM_HEADS + h
        line_v[pl.ds(k0 + c * L, L)] = plsc.load_gather(table_v, [idx])
        return carry

    lax.fori_loop(0, HALF_W // L + 1, line_body, 0)

    def shift_body(c, carry):
        base = k0 + c * L
        kidx = base + lane
        for r in range(8):
            shift_v[r, pl.ds(base - k0, L)] = plsc.load_gather(
                line_v, [kidx + 7 - r])
        return carry

    lax.fori_loop(0, HALF_W // L, shift_body, 0)

    pltpu.sync_copy(shift_v, shift_hbm.at[h, :, pl.ds(k0, HALF_W)])


def _tc_stripes_body(shift_ref, out_ref, buf, sem):
    h = pl.program_id(0)
    v = pl.program_id(1)
    p = h * 16 + v
    par = jnp.bitwise_and(p, 7)
    nlast = NUM_HEADS * 16 - 1

    def _drain(par_idx):
        pltpu.make_async_copy(
            out_ref.at[0, 1, pl.ds(0, 128), :],
            out_ref.at[0, 0, pl.ds(0, 128), :],
            sem.at[par_idx],
        ).wait()

    rolled = pltpu.roll(shift_ref[0], WIDTH - 8 * v, axis=1)

    @pl.when(p >= 8)
    def _():
        _drain(par)

    buf[par] = rolled

    for a in range(16):
        i = (S - 8) - 128 * a - 8 * v
        pltpu.make_async_copy(
            buf.at[par, :, pl.ds(128 * a, S)],
            out_ref.at[0, h, pl.ds(pl.multiple_of(i, 8), 8), :],
            sem.at[par],
        ).start()

    @pl.when(p == nlast)
    def _():
        for slot in range(8):
            _drain(slot)


def kernel(seq_len, table):
    del seq_len
    table_flat = table.reshape(NUM_BUCKETS * NUM_HEADS)

    sc_launch = pl.kernel(
        _sc_lines_body,
        mesh=plsc.VectorSubcoreMesh(core_axis_name="c", subcore_axis_name="s"),
        out_type=jax.ShapeDtypeStruct((NUM_HEADS, 8, WIDTH), jnp.float32),
        compiler_params=pltpu.CompilerParams(needs_layout_passes=False),
        scratch_types=[
            pltpu.VMEM((NUM_BUCKETS * NUM_HEADS,), jnp.float32),
            pltpu.VMEM((LINE_PAD,), jnp.float32),
            pltpu.VMEM((8, HALF_W), jnp.float32),
            pltpu.SemaphoreType.DMA,
        ],
    )
    shift = sc_launch(table_flat)

    out = pl.pallas_call(
        _tc_stripes_body,
        grid=(NUM_HEADS, 16),
        in_specs=[
            pl.BlockSpec((1, 8, WIDTH), lambda h, v: (h, 0, 0)),
        ],
        out_specs=pl.BlockSpec(memory_space=pl.ANY),
        out_shape=jax.ShapeDtypeStruct((1, NUM_HEADS, S, S), jnp.float32),
        scratch_shapes=[
            pltpu.VMEM((8, 8, WIDTH), jnp.float32),
            pltpu.SemaphoreType.DMA((8,)),
        ],
        compiler_params=pltpu.CompilerParams(
            dimension_semantics=("arbitrary", "arbitrary")),
    )(shift)
    return out
